# baseline (speedup 1.0000x reference)
import jax
import jax.numpy as jnp
import numpy as np
from jax import lax
from jax.experimental import pallas as pl
from jax.experimental.pallas import tpu as pltpu
from jax.experimental.pallas import tpu_sc as plsc

N = 10000
D = 128
B = 1024
H1 = 256
NN1 = 10
NN2 = 25
NW = 32
SEEDS_PER = B // NW
S1_PER = SEEDS_PER * NN1
NP = 16
OUTER = S1_PER // NP
ROWS = NP * NN2
VL = 16
MAXDEG = 32
DCOL = MAXDEG
ADJW = 48

_COLS0 = np.array([17, 27, 1, 3, 28, 19, 9, 11, 31, 5], np.int32)
_COLS1 = np.array([2, 15, 10, 25, 28, 0, 4, 21, 11, 20, 17, 12, 19,
                   22, 18, 16, 27, 5, 23, 26, 7, 29, 9, 13, 6], np.int32)

_COLSTAB = np.zeros(128, np.int32)
_COLSTAB[:NN1] = _COLS0
_COLSTAB[NN1:NN1 + NN2] = _COLS1


def _prep_body(f_ref, adjc_ref, gbfi_ref):
    dp = jax.lax.bitcast_convert_type(adjc_ref[:, DCOL:DCOL + 1],
                                      jnp.float32)
    g = f_ref[...] / dp
    a = g[:, :D // 2].astype(jnp.bfloat16).astype(jnp.float32)
    b = g[:, D // 2:].astype(jnp.bfloat16).astype(jnp.float32)
    ai = jax.lax.bitcast_convert_type(a, jnp.int32)
    bi = jax.lax.bitcast_convert_type(b, jnp.int32)
    gbfi_ref[...] = jax.lax.shift_right_logical(ai, 16) | (
        (bi >> 16) << 16)


def _prep(features, adjC):
    blk = N // 10
    return pl.pallas_call(
        _prep_body,
        grid=(10,),
        in_specs=[pl.BlockSpec((blk, D), lambda i: (i, 0)),
                  pl.BlockSpec((blk, ADJW), lambda i: (i, 0))],
        out_specs=pl.BlockSpec((blk, D // 2), lambda i: (i, 0)),
        out_shape=jax.ShapeDtypeStruct((N, D // 2), jnp.int32),
    )(features, adjC)


def _sc_body(colstab, adjC, batch, gbfi,
             g0o, d0o, g1o, d1o, a0o, a1o,
             colsvm, bidx, amat, s1loc, s1g, s2ids0, s2ids1, rows0, rows1,
             la0, la1, aggb, dbuf, agg0buf, bmsem, rsem0, rsem1,
             lasem, osem0, osem1):
    wid = lax.axis_index("s") * 2 + lax.axis_index("c")
    iota = lax.iota(jnp.int32, VL)
    dcol = iota * 0 + DCOL

    pltpu.sync_copy(colstab, colsvm)
    pltpu.sync_copy(batch.at[pl.ds(wid * SEEDS_PER, SEEDS_PER)], bidx)
    pltpu.sync_copy(adjC.at[bidx], amat)
    for k in range(S1_PER // VL):
        t = iota + (k * VL)
        p = t // NN1
        j = plsc.load_gather(colsvm, [t - p * NN1])
        vals = plsc.load_gather(amat, [p, j])
        s1loc[k, :] = vals
        s1g[k // 5, pl.ds((k % 5) * VL, VL)] = vals

    for k in range(SEEDS_PER // VL):
        dv = plsc.load_gather(amat, [iota + k * VL, dcol])
        dbuf[pl.ds(k * VL, VL)] = plsc.bitcast(dv, jnp.float32)
    pltpu.sync_copy(dbuf.at[pl.ds(0, SEEDS_PER)],
                    d0o.at[pl.ds(wid * SEEDS_PER, SEEDS_PER)])

    pltpu.sync_copy(gbfi.at[bidx], la0.at[pl.ds(0, SEEDS_PER)])
    pltpu.sync_copy(la0.at[pl.ds(0, SEEDS_PER)],
                    g0o.at[pl.ds(wid * SEEDS_PER, SEEDS_PER)])

    rbufs = (rows0, rows1)
    labufs = (la0, la1)
    rsems = (rsem0, rsem1)
    for q in range(2):
        pltpu.async_copy(gbfi.at[s1g.at[q]], labufs[q % 2], rsems[q % 2])
    for q in range(4):
        rb = labufs[q % 2]
        pltpu.make_async_copy(gbfi.at[s1g.at[q]], rb, rsems[q % 2]).wait()
        pltpu.async_copy(rb, g1o.at[pl.ds(wid * S1_PER + q * 80, 80)],
                         lasem)

        @pl.loop(0, 8)
        def _pool0(s, q=q):
            base = s * NN1
            buf = labufs[q % 2]

            def bfl(r, c):
                return plsc.bitcast(buf[r, pl.ds(c * VL, VL)], jnp.bfloat16)
            accs = [bfl(base, c) for c in range(4)]
            for jj in range(1, NN1):
                for c in range(4):
                    accs[c] = accs[c] + bfl(base + jj, c)
            for c in range(4):
                agg0buf[q * 8 + s, pl.ds(c * VL, VL)] = plsc.bitcast(
                    accs[c], jnp.int32)

        if q + 2 < 4:
            pltpu.make_async_copy(
                rb, g1o.at[pl.ds(wid * S1_PER + q * 80, 80)], lasem).wait()
            pltpu.async_copy(gbfi.at[s1g.at[q + 2]], labufs[q % 2],
                             rsems[q % 2])

    for q in (2, 3):
        pltpu.make_async_copy(
            labufs[q % 2], g1o.at[pl.ds(wid * S1_PER + q * 80, 80)],
            lasem).wait()
    pltpu.sync_copy(agg0buf, a0o.at[pl.ds(wid * SEEDS_PER, SEEDS_PER)])

    sbufs = (s2ids0, s2ids1)

    def _bmat_ref(b):
        return amat.at[pl.ds(b * NP, NP)]

    def _fire_bmat(o, b):
        pltpu.async_copy(adjC.at[s1loc.at[o]], _bmat_ref(b), bmsem)

    def _wait_bmat(o, b):
        pltpu.make_async_copy(adjC.at[s1loc.at[o]], _bmat_ref(b),
                              bmsem).wait()

    def _extract_fire(o, b):
        bmat = _bmat_ref(b)
        dv = plsc.load_gather(bmat, [iota, dcol])
        dbuf[pl.ds(o * VL, VL)] = plsc.bitcast(dv, jnp.float32)
        for m in range(NP * NN2 // VL):
            t = iota + (m * VL)
            p = t // NN2
            j = plsc.load_gather(colsvm, [t - p * NN2 + NN1])
            vals = plsc.load_gather(bmat, [p, j])
            sbufs[b][m // 5, pl.ds((m % 5) * VL, VL)] = vals
        for qq in range(5):
            pltpu.async_copy(gbfi.at[sbufs[b].at[qq]],
                             rbufs[b].at[pl.ds(qq * 80, 80)], rsems[b])

    osems = (osem0, osem1)

    def _pool_out(o, b):
        for qq in range(5):
            pltpu.make_async_copy(gbfi.at[sbufs[b].at[qq]],
                                  rbufs[b].at[pl.ds(qq * 80, 80)],
                                  rsems[b]).wait()
        buf = rbufs[b]
        agg = aggb.at[pl.ds(b * NP, NP)]

        @pl.when(o >= 2)
        def _():
            pltpu.make_async_copy(
                agg, a1o.at[pl.ds(wid * S1_PER + (o - 2) * NP, NP)],
                osems[b]).wait()

        @pl.loop(0, NP)
        def _pool1(p):
            base = p * NN2
            def bfld(r, c):
                return plsc.bitcast(buf[r, pl.ds(c * VL, VL)], jnp.bfloat16)
            accs = [bfld(base, c) for c in range(4)]
            for jj in range(1, NN2):
                for c in range(4):
                    accs[c] = accs[c] + bfld(base + jj, c)
            for c in range(4):
                agg[p, pl.ds(c * VL, VL)] = plsc.bitcast(accs[c], jnp.int32)

        pltpu.async_copy(agg, a1o.at[pl.ds(wid * S1_PER + o * NP, NP)],
                         osems[b])

    s1loc[OUTER, :] = jnp.zeros((VL,), jnp.int32)
    _fire_bmat(0, 0)

    @pl.loop(0, OUTER // 2)
    def _outer(u):
        o0 = u * 2
        _wait_bmat(o0, 0)
        _extract_fire(o0, 0)
        _fire_bmat(o0 + 1, 1)

        @pl.when(u > 0)
        def _():
            _pool_out(o0 - 1, 1)

        o1 = o0 + 1
        _wait_bmat(o1, 1)
        _extract_fire(o1, 1)
        _fire_bmat(o1 + 1, 0)
        _pool_out(o0, 0)

    _wait_bmat(OUTER, 0)
    _pool_out(OUTER - 1, 1)
    pltpu.make_async_copy(
        aggb.at[pl.ds(0, NP)],
        a1o.at[pl.ds(wid * S1_PER + (OUTER - 2) * NP, NP)], osem0).wait()
    pltpu.make_async_copy(
        aggb.at[pl.ds(NP, NP)],
        a1o.at[pl.ds(wid * S1_PER + (OUTER - 1) * NP, NP)], osem1).wait()
    pltpu.sync_copy(dbuf, d1o.at[pl.ds(wid * 384, 384)])


def _sc_gather(adjC, batch, gbfi):
    mesh = plsc.VectorSubcoreMesh(core_axis_name="c", subcore_axis_name="s")
    kern = pl.kernel(
        _sc_body,
        compiler_params=pltpu.CompilerParams(
            needs_layout_passes=False, use_tc_tiling_on_sc=False),
        out_type=(
            jax.ShapeDtypeStruct((B, D // 2), jnp.int32),
            jax.ShapeDtypeStruct((B,), jnp.float32),
            jax.ShapeDtypeStruct((B * NN1, D // 2), jnp.int32),
            jax.ShapeDtypeStruct((NW * 384,), jnp.float32),
            jax.ShapeDtypeStruct((B, D // 2), jnp.int32),
            jax.ShapeDtypeStruct((B * NN1, D // 2), jnp.int32),
        ),
        mesh=mesh,
        scratch_types=[
            pltpu.VMEM((128,), jnp.int32),
            pltpu.VMEM((SEEDS_PER,), jnp.int32),
            pltpu.VMEM((SEEDS_PER, ADJW), jnp.int32),
            pltpu.VMEM((OUTER + 1, NP), jnp.int32),
            pltpu.VMEM((4, 80), jnp.int32),
            pltpu.VMEM((5, 80), jnp.int32),
            pltpu.VMEM((5, 80), jnp.int32),
            pltpu.VMEM((ROWS, D // 2), jnp.int32),
            pltpu.VMEM((ROWS, D // 2), jnp.int32),
            pltpu.VMEM((80, D // 2), jnp.int32),
            pltpu.VMEM((80, D // 2), jnp.int32),
            pltpu.VMEM((2 * NP, D // 2), jnp.int32),
            pltpu.VMEM((384,), jnp.float32),
            pltpu.VMEM((SEEDS_PER, D // 2), jnp.int32),
            pltpu.SemaphoreType.DMA,
            pltpu.SemaphoreType.DMA,
            pltpu.SemaphoreType.DMA,
            pltpu.SemaphoreType.DMA,
            pltpu.SemaphoreType.DMA,
            pltpu.SemaphoreType.DMA,
        ],
    )
    return kern(jnp.asarray(_COLSTAB), adjC, batch, gbfi)


def _expand(dr, n):
    e = jnp.eye(VL, dtype=jnp.float32)
    t3 = dr[:, :, None] * e[None, :, :]
    return jnp.sum(t3.reshape(n, VL), axis=1, keepdims=True)


def _unpack(p):
    lo = jax.lax.bitcast_convert_type(p << 16, jnp.float32)
    hi = jax.lax.bitcast_convert_type((p >> 16) << 16, jnp.float32)
    return jnp.concatenate([lo, hi], axis=1)


def _main_body(g0, d0, g1, d1, a0, a1, eps, W0, Wm, Ws, Wp, out):
    hp = None
    d0c = _expand(d0[...], B)
    d1c = _expand(d1[...], B * NN1)
    x1 = _unpack(g1[...]) * d1c + _unpack(a1[...]) * (1.0 / NN2)
    h1 = jnp.tanh(jnp.dot(x1, W0[...], precision=hp))
    wh = (h1 / d1c) * (1.0 / NN1)
    aggh = jnp.sum(wh.reshape(B, NN1, H1), axis=1)
    x0 = _unpack(g0[...]) * d0c + _unpack(a0[...]) * (1.0 / NN1)
    h0 = jnp.tanh(jnp.dot(x0, W0[...], precision=hp))
    u = h0 + aggh
    zm = jnp.dot(u, Wm[...], precision=hp)
    zs = jnp.dot(u, Ws[...], precision=hp)
    z = zm + eps[...] * jnp.exp(zs)
    out[...] = jnp.dot(z, Wp[...], precision=hp)


def _main(g0, d0, g1, d1, a0, a1, eps, W0, Wm, Ws, Wp):
    return pl.pallas_call(
        _main_body,
        out_shape=jax.ShapeDtypeStruct((B, 16), jnp.float32),
    )(g0, d0, g1, d1, a0, a1, eps, W0, Wm, Ws, Wp)


def kernel(features, adj_info, degrees, batch, eps, W0, Wm, Ws, Wp):
    adj32 = adj_info.astype(jnp.int32)
    batch32 = batch.astype(jnp.int32)
    dbits = jax.lax.bitcast_convert_type(degrees + 1.0, jnp.int32)
    adjC = jnp.concatenate(
        [adj32, dbits.reshape(N, 1),
         jnp.zeros((N, ADJW - MAXDEG - 1), jnp.int32)], axis=1)

    gbfi = _prep(features, adjC)
    g0, d0, g1, d1, a0, a1 = _sc_gather(adjC, batch32, gbfi)
    d0 = d0.reshape(B // VL, VL)
    d1 = d1.reshape(NW, 384)[:, :S1_PER].reshape(B * NN1 // VL, VL)
    return _main(g0, d0, g1, d1, a0, a1, eps, W0, Wm, Ws, Wp)


# speedup vs baseline: 1.0192x; 1.0192x over previous
import jax
import jax.numpy as jnp
import numpy as np
from jax import lax
from jax.experimental import pallas as pl
from jax.experimental.pallas import tpu as pltpu
from jax.experimental.pallas import tpu_sc as plsc

N = 10000
D = 128
B = 1024
H1 = 256
NN1 = 10
NN2 = 25
NW = 32
SEEDS_PER = B // NW
S1_PER = SEEDS_PER * NN1
NP = 16
OUTER = S1_PER // NP
ROWS = NP * NN2
VL = 16
MAXDEG = 32
DCOL = MAXDEG
ADJW = 48

_COLS0 = np.array([17, 27, 1, 3, 28, 19, 9, 11, 31, 5], np.int32)
_COLS1 = np.array([2, 15, 10, 25, 28, 0, 4, 21, 11, 20, 17, 12, 19,
                   22, 18, 16, 27, 5, 23, 26, 7, 29, 9, 13, 6], np.int32)

_COLSTAB = np.zeros(128, np.int32)
_COLSTAB[:NN1] = _COLS0
_COLSTAB[NN1:NN1 + NN2] = _COLS1


def _prep_body(f_ref, adjc_ref, gbfi_ref):
    dp = jax.lax.bitcast_convert_type(adjc_ref[:, DCOL:DCOL + 1],
                                      jnp.float32)
    g = f_ref[...] / dp
    a = g[:, :D // 2].astype(jnp.bfloat16).astype(jnp.float32)
    b = g[:, D // 2:].astype(jnp.bfloat16).astype(jnp.float32)
    ai = jax.lax.bitcast_convert_type(a, jnp.int32)
    bi = jax.lax.bitcast_convert_type(b, jnp.int32)
    gbfi_ref[...] = jax.lax.shift_right_logical(ai, 16) | (
        (bi >> 16) << 16)


def _prep(features, adjC):
    return pl.pallas_call(
        _prep_body,
        out_shape=jax.ShapeDtypeStruct((N, D // 2), jnp.int32),
    )(features, adjC)


def _sc_body(colstab, adjC, batch, gbfi,
             g0o, d0o, g1o, d1o, a0o, a1o,
             colsvm, bidx, amat, s1loc, s1g, s2ids0, s2ids1, rows0, rows1,
             la0, la1, aggb, dbuf, agg0buf, bmsem, rsem0, rsem1,
             lasem, osem0, osem1):
    wid = lax.axis_index("s") * 2 + lax.axis_index("c")
    iota = lax.iota(jnp.int32, VL)
    dcol = iota * 0 + DCOL

    pltpu.sync_copy(colstab, colsvm)
    pltpu.sync_copy(batch.at[pl.ds(wid * SEEDS_PER, SEEDS_PER)], bidx)
    pltpu.sync_copy(adjC.at[bidx], amat)
    for k in range(S1_PER // VL):
        t = iota + (k * VL)
        p = t // NN1
        j = plsc.load_gather(colsvm, [t - p * NN1])
        vals = plsc.load_gather(amat, [p, j])
        s1loc[k, :] = vals
        s1g[k // 5, pl.ds((k % 5) * VL, VL)] = vals

    for k in range(SEEDS_PER // VL):
        dv = plsc.load_gather(amat, [iota + k * VL, dcol])
        dbuf[pl.ds(k * VL, VL)] = plsc.bitcast(dv, jnp.float32)
    pltpu.sync_copy(dbuf.at[pl.ds(0, SEEDS_PER)],
                    d0o.at[pl.ds(wid * SEEDS_PER, SEEDS_PER)])

    pltpu.sync_copy(gbfi.at[bidx], la0.at[pl.ds(0, SEEDS_PER)])
    pltpu.sync_copy(la0.at[pl.ds(0, SEEDS_PER)],
                    g0o.at[pl.ds(wid * SEEDS_PER, SEEDS_PER)])

    rbufs = (rows0, rows1)
    labufs = (la0, la1)
    rsems = (rsem0, rsem1)
    for q in range(2):
        pltpu.async_copy(gbfi.at[s1g.at[q]], labufs[q % 2], rsems[q % 2])
    for q in range(4):
        rb = labufs[q % 2]
        pltpu.make_async_copy(gbfi.at[s1g.at[q]], rb, rsems[q % 2]).wait()
        pltpu.async_copy(rb, g1o.at[pl.ds(wid * S1_PER + q * 80, 80)],
                         lasem)

        @pl.loop(0, 8)
        def _pool0(s, q=q):
            base = s * NN1
            buf = labufs[q % 2]

            def bfl(r, c):
                return plsc.bitcast(buf[r, pl.ds(c * VL, VL)], jnp.bfloat16)
            accs = [bfl(base, c) for c in range(4)]
            for jj in range(1, NN1):
                for c in range(4):
                    accs[c] = accs[c] + bfl(base + jj, c)
            for c in range(4):
                agg0buf[q * 8 + s, pl.ds(c * VL, VL)] = plsc.bitcast(
                    accs[c], jnp.int32)

        if q + 2 < 4:
            pltpu.make_async_copy(
                rb, g1o.at[pl.ds(wid * S1_PER + q * 80, 80)], lasem).wait()
            pltpu.async_copy(gbfi.at[s1g.at[q + 2]], labufs[q % 2],
                             rsems[q % 2])

    for q in (2, 3):
        pltpu.make_async_copy(
            labufs[q % 2], g1o.at[pl.ds(wid * S1_PER + q * 80, 80)],
            lasem).wait()
    pltpu.sync_copy(agg0buf, a0o.at[pl.ds(wid * SEEDS_PER, SEEDS_PER)])

    sbufs = (s2ids0, s2ids1)

    def _bmat_ref(b):
        return amat.at[pl.ds(b * NP, NP)]

    def _fire_bmat(o, b):
        pltpu.async_copy(adjC.at[s1loc.at[o]], _bmat_ref(b), bmsem)

    def _wait_bmat(o, b):
        pltpu.make_async_copy(adjC.at[s1loc.at[o]], _bmat_ref(b),
                              bmsem).wait()

    def _extract_fire(o, b):
        bmat = _bmat_ref(b)
        dv = plsc.load_gather(bmat, [iota, dcol])
        dbuf[pl.ds(o * VL, VL)] = plsc.bitcast(dv, jnp.float32)
        for m in range(NP * NN2 // VL):
            t = iota + (m * VL)
            p = t // NN2
            j = plsc.load_gather(colsvm, [t - p * NN2 + NN1])
            vals = plsc.load_gather(bmat, [p, j])
            sbufs[b][m // 5, pl.ds((m % 5) * VL, VL)] = vals
        for qq in range(5):
            pltpu.async_copy(gbfi.at[sbufs[b].at[qq]],
                             rbufs[b].at[pl.ds(qq * 80, 80)], rsems[b])

    osems = (osem0, osem1)

    def _pool_out(o, b):
        for qq in range(5):
            pltpu.make_async_copy(gbfi.at[sbufs[b].at[qq]],
                                  rbufs[b].at[pl.ds(qq * 80, 80)],
                                  rsems[b]).wait()
        buf = rbufs[b]
        agg = aggb.at[pl.ds(b * NP, NP)]

        @pl.when(o >= 2)
        def _():
            pltpu.make_async_copy(
                agg, a1o.at[pl.ds(wid * S1_PER + (o - 2) * NP, NP)],
                osems[b]).wait()

        @pl.loop(0, NP)
        def _pool1(p):
            base = p * NN2
            def bfld(r, c):
                return plsc.bitcast(buf[r, pl.ds(c * VL, VL)], jnp.bfloat16)
            accs = [bfld(base, c) for c in range(4)]
            for jj in range(1, NN2):
                for c in range(4):
                    accs[c] = accs[c] + bfld(base + jj, c)
            for c in range(4):
                agg[p, pl.ds(c * VL, VL)] = plsc.bitcast(accs[c], jnp.int32)

        pltpu.async_copy(agg, a1o.at[pl.ds(wid * S1_PER + o * NP, NP)],
                         osems[b])

    s1loc[OUTER, :] = jnp.zeros((VL,), jnp.int32)
    _fire_bmat(0, 0)

    @pl.loop(0, OUTER // 2)
    def _outer(u):
        o0 = u * 2
        _wait_bmat(o0, 0)
        _extract_fire(o0, 0)
        _fire_bmat(o0 + 1, 1)

        @pl.when(u > 0)
        def _():
            _pool_out(o0 - 1, 1)

        o1 = o0 + 1
        _wait_bmat(o1, 1)
        _extract_fire(o1, 1)
        _fire_bmat(o1 + 1, 0)
        _pool_out(o0, 0)

    _wait_bmat(OUTER, 0)
    _pool_out(OUTER - 1, 1)
    pltpu.make_async_copy(
        aggb.at[pl.ds(0, NP)],
        a1o.at[pl.ds(wid * S1_PER + (OUTER - 2) * NP, NP)], osem0).wait()
    pltpu.make_async_copy(
        aggb.at[pl.ds(NP, NP)],
        a1o.at[pl.ds(wid * S1_PER + (OUTER - 1) * NP, NP)], osem1).wait()
    pltpu.sync_copy(dbuf, d1o.at[pl.ds(wid * 384, 384)])


def _sc_gather(adjC, batch, gbfi):
    mesh = plsc.VectorSubcoreMesh(core_axis_name="c", subcore_axis_name="s")
    kern = pl.kernel(
        _sc_body,
        compiler_params=pltpu.CompilerParams(
            needs_layout_passes=False, use_tc_tiling_on_sc=False),
        out_type=(
            jax.ShapeDtypeStruct((B, D // 2), jnp.int32),
            jax.ShapeDtypeStruct((B,), jnp.float32),
            jax.ShapeDtypeStruct((B * NN1, D // 2), jnp.int32),
            jax.ShapeDtypeStruct((NW * 384,), jnp.float32),
            jax.ShapeDtypeStruct((B, D // 2), jnp.int32),
            jax.ShapeDtypeStruct((B * NN1, D // 2), jnp.int32),
        ),
        mesh=mesh,
        scratch_types=[
            pltpu.VMEM((128,), jnp.int32),
            pltpu.VMEM((SEEDS_PER,), jnp.int32),
            pltpu.VMEM((SEEDS_PER, ADJW), jnp.int32),
            pltpu.VMEM((OUTER + 1, NP), jnp.int32),
            pltpu.VMEM((4, 80), jnp.int32),
            pltpu.VMEM((5, 80), jnp.int32),
            pltpu.VMEM((5, 80), jnp.int32),
            pltpu.VMEM((ROWS, D // 2), jnp.int32),
            pltpu.VMEM((ROWS, D // 2), jnp.int32),
            pltpu.VMEM((80, D // 2), jnp.int32),
            pltpu.VMEM((80, D // 2), jnp.int32),
            pltpu.VMEM((2 * NP, D // 2), jnp.int32),
            pltpu.VMEM((384,), jnp.float32),
            pltpu.VMEM((SEEDS_PER, D // 2), jnp.int32),
            pltpu.SemaphoreType.DMA,
            pltpu.SemaphoreType.DMA,
            pltpu.SemaphoreType.DMA,
            pltpu.SemaphoreType.DMA,
            pltpu.SemaphoreType.DMA,
            pltpu.SemaphoreType.DMA,
        ],
    )
    return kern(jnp.asarray(_COLSTAB), adjC, batch, gbfi)


def _expand(dr, n):
    e = jnp.eye(VL, dtype=jnp.float32)
    t3 = dr[:, :, None] * e[None, :, :]
    return jnp.sum(t3.reshape(n, VL), axis=1, keepdims=True)


def _unpack(p):
    lo = jax.lax.bitcast_convert_type(p << 16, jnp.float32)
    hi = jax.lax.bitcast_convert_type((p >> 16) << 16, jnp.float32)
    return jnp.concatenate([lo, hi], axis=1)


def _main_body(g0, d0, g1, d1, a0, a1, eps, W0, Wm, Ws, Wp, out):
    hp = None
    d0c = _expand(d0[...], B)
    d1c = _expand(d1[...], B * NN1)
    x1 = _unpack(g1[...]) * d1c + _unpack(a1[...]) * (1.0 / NN2)
    h1 = jnp.tanh(jnp.dot(x1, W0[...], precision=hp))
    wh = (h1 / d1c) * (1.0 / NN1)
    aggh = jnp.sum(wh.reshape(B, NN1, H1), axis=1)
    x0 = _unpack(g0[...]) * d0c + _unpack(a0[...]) * (1.0 / NN1)
    h0 = jnp.tanh(jnp.dot(x0, W0[...], precision=hp))
    u = h0 + aggh
    zm = jnp.dot(u, Wm[...], precision=hp)
    zs = jnp.dot(u, Ws[...], precision=hp)
    z = zm + eps[...] * jnp.exp(zs)
    out[...] = jnp.dot(z, Wp[...], precision=hp)


def _main(g0, d0, g1, d1, a0, a1, eps, W0, Wm, Ws, Wp):
    return pl.pallas_call(
        _main_body,
        out_shape=jax.ShapeDtypeStruct((B, 16), jnp.float32),
    )(g0, d0, g1, d1, a0, a1, eps, W0, Wm, Ws, Wp)


def kernel(features, adj_info, degrees, batch, eps, W0, Wm, Ws, Wp):
    adj32 = adj_info.astype(jnp.int32)
    batch32 = batch.astype(jnp.int32)
    dbits = jax.lax.bitcast_convert_type(degrees + 1.0, jnp.int32)
    adjC = jnp.concatenate(
        [adj32, dbits.reshape(N, 1),
         jnp.zeros((N, ADJW - MAXDEG - 1), jnp.int32)], axis=1)

    gbfi = _prep(features, adjC)
    g0, d0, g1, d1, a0, a1 = _sc_gather(adjC, batch32, gbfi)
    d0 = d0.reshape(B // VL, VL)
    d1 = d1.reshape(NW, 384)[:, :S1_PER].reshape(B * NN1 // VL, VL)
    return _main(g0, d0, g1, d1, a0, a1, eps, W0, Wm, Ws, Wp)
